# baseline (speedup 1.0000x reference)
import functools

import jax
import jax.numpy as jnp
from jax import lax
from jax.experimental import pallas as pl
from jax.experimental.pallas import tpu as pltpu
from jax.experimental.pallas import tpu_sc as plsc

_NUM_CORES = 2
_NUM_SUBCORES = 16
_LANES = 16
_NW = _NUM_CORES * _NUM_SUBCORES


def _router_kernel(x_ref, rw_ref, idx_ref, gate_ref, *, experts, block):
    x = x_ref[0]
    logits = jnp.dot(x, rw_ref[...], preferred_element_type=jnp.float32)
    mx = jnp.max(logits, axis=1, keepdims=True)
    ex = jnp.exp(logits - mx)
    s = jnp.sum(ex, axis=1, keepdims=True)
    probs = ex / s
    pm = jnp.max(probs, axis=1, keepdims=True)
    col = jax.lax.broadcasted_iota(jnp.int32, probs.shape, 1)
    first = jnp.min(jnp.where(probs == pm, col, experts), axis=1,
                    keepdims=True)
    r2 = jax.lax.broadcasted_iota(jnp.int32, (block, block), 0)
    c2 = jax.lax.broadcasted_iota(jnp.int32, (block, block), 1)
    eye = (r2 == c2).astype(jnp.float32)
    gate_row = jax.lax.dot_general(pm, eye, (((0,), (0,)), ((), ())),
                                   preferred_element_type=jnp.float32)
    idx_row = jax.lax.dot_general(first.astype(jnp.float32), eye,
                                  (((0,), (0,)), ((), ())),
                                  preferred_element_type=jnp.float32)
    idx_ref[...] = idx_row.astype(jnp.int32)[None]
    gate_ref[...] = gate_row[None]


def _expert_kernel(x_ref, ew_ref, eb_ref, sp_ref, out_ref):
    sp = sp_ref[...]
    s_row = jnp.sum(sp, axis=0, keepdims=True)
    ones = jnp.ones((1, 1), dtype=jnp.float32)
    s_col = jax.lax.dot_general(s_row, ones, (((0,), (0,)), ((), ())),
                                preferred_element_type=jnp.float32)
    x = x_ref[0]
    y = jax.lax.dot_general(x, ew_ref[...], (((1,), (1,)), ((), ())),
                            preferred_element_type=jnp.float32)
    out_ref[...] = ((y + eb_ref[...]) * s_col)[None]


def _make_sc_routing(T, experts, capacity):
    epw = experts // _NW
    n_vec = T // _LANES

    mesh = plsc.VectorSubcoreMesh(core_axis_name="c", subcore_axis_name="s",
                                  num_cores=_NUM_CORES)

    @functools.partial(
        pl.kernel,
        mesh=mesh,
        compiler_params=pltpu.CompilerParams(needs_layout_passes=False),
        out_type=jax.ShapeDtypeStruct((_NW, T), jnp.float32),
        scratch_types=[
            pltpu.VMEM((T,), jnp.int32),
            pltpu.VMEM((T,), jnp.float32),
            pltpu.VMEM((T,), jnp.float32),
        ],
    )
    def sc_routing(idx_hbm, gate_hbm, out_hbm, idx_v, gate_v, acc_v):
        c = lax.axis_index("c")
        s = lax.axis_index("s")
        wid = c * _NUM_SUBCORES + s
        e0v = jnp.full((_LANES,), wid * epw, jnp.int32)
        e1v = e0v + jnp.full((_LANES,), 1, jnp.int32)
        capv = jnp.full((_LANES,), capacity, jnp.int32)
        zerov = jnp.zeros((_LANES,), jnp.float32)
        onei = jnp.full((_LANES,), 1, jnp.int32)
        zeroi = jnp.zeros((_LANES,), jnp.int32)

        pltpu.sync_copy(idx_hbm, idx_v)
        pltpu.sync_copy(gate_hbm, gate_v)

        zero_base = jnp.zeros((_LANES,), jnp.int32)

        def body(i, carry):
            base0, base1 = carry
            off = pl.multiple_of(i * _LANES, _LANES)
            iv = idx_v[pl.ds(off, _LANES)]
            gv = gate_v[pl.ds(off, _LANES)]

            m0 = iv == e0v
            c0 = plsc.cumsum(jnp.where(m0, onei, zeroi))
            k0 = m0 & ((base0 + c0) <= capv)
            base0 = base0 + plsc.all_reduce_population_count(m0)

            m1 = iv == e1v
            c1 = plsc.cumsum(jnp.where(m1, onei, zeroi))
            k1 = m1 & ((base1 + c1) <= capv)
            base1 = base1 + plsc.all_reduce_population_count(m1)

            acc_v[pl.ds(off, _LANES)] = jnp.where(k0 | k1, gv, zerov)
            return base0, base1

        lax.fori_loop(0, n_vec, body, (zero_base, zero_base))
        pltpu.sync_copy(acc_v, out_hbm.at[wid])

    return sc_routing


def kernel(x, router_w, expert_w, expert_b):
    B, S, H = x.shape
    E = router_w.shape[1]
    T = B * S
    capacity = T // E * 2
    block = 256
    grid = T // block

    x3 = x.reshape(grid, block, H)
    eb = expert_b.reshape(1, H)

    idx3, gate3 = pl.pallas_call(
        functools.partial(_router_kernel, experts=E, block=block),
        grid=(grid,),
        in_specs=[
            pl.BlockSpec((1, block, H), lambda i: (i, 0, 0)),
            pl.BlockSpec((H, E), lambda i: (0, 0)),
        ],
        out_specs=[
            pl.BlockSpec((1, 1, block), lambda i: (i, 0, 0)),
            pl.BlockSpec((1, 1, block), lambda i: (i, 0, 0)),
        ],
        out_shape=[
            jax.ShapeDtypeStruct((grid, 1, block), jnp.int32),
            jax.ShapeDtypeStruct((grid, 1, block), jnp.float32),
        ],
    )(x3, router_w)

    sc_routing = _make_sc_routing(T, E, capacity)
    parts = sc_routing(idx3.reshape(T), gate3.reshape(T))

    out = pl.pallas_call(
        _expert_kernel,
        grid=(grid,),
        in_specs=[
            pl.BlockSpec((1, block, H), lambda i: (i, 0, 0)),
            pl.BlockSpec((H, H), lambda i: (0, 0)),
            pl.BlockSpec((1, H), lambda i: (0, 0)),
            pl.BlockSpec((_NW, block), lambda i: (0, i)),
        ],
        out_specs=pl.BlockSpec((1, block, H), lambda i: (i, 0, 0)),
        out_shape=jax.ShapeDtypeStruct((grid, block, H), jnp.float32),
    )(x3, expert_w, eb, parts)
    return out.reshape(B, S, H)


# speedup vs baseline: 1.0565x; 1.0565x over previous
import functools

import jax
import jax.numpy as jnp
from jax import lax
from jax.experimental import pallas as pl
from jax.experimental.pallas import tpu as pltpu
from jax.experimental.pallas import tpu_sc as plsc

_NUM_CORES = 2
_NUM_SUBCORES = 16
_LANES = 16
_NW = _NUM_CORES * _NUM_SUBCORES


def _router_kernel(x_ref, rw_ref, idx_ref, gate_ref, *, experts, block):
    x = x_ref[0].astype(jnp.bfloat16)
    rw = rw_ref[...].astype(jnp.bfloat16)
    lt = jax.lax.dot_general(rw, x, (((0,), (1,)), ((), ())),
                             preferred_element_type=jnp.float32)
    mx = jnp.max(lt, axis=0, keepdims=True)
    ex = jnp.exp(lt - mx)
    s = jnp.sum(ex, axis=0, keepdims=True)
    probs = ex / s
    pm = jnp.max(probs, axis=0, keepdims=True)
    row = jax.lax.broadcasted_iota(jnp.int32, probs.shape, 0)
    first = jnp.min(jnp.where(probs == pm, row, experts), axis=0,
                    keepdims=True)
    idx_ref[...] = first[None]
    gate_ref[...] = pm[None]


def _expert_kernel(x_ref, ew_ref, eb_ref, sp_ref, out_ref):
    sp = sp_ref[...]
    s_row = jnp.sum(sp, axis=0, keepdims=True)
    ones = jnp.ones((1, 1), dtype=jnp.float32)
    s_col = jax.lax.dot_general(s_row, ones, (((0,), (0,)), ((), ())),
                                preferred_element_type=jnp.float32)
    x = x_ref[0].astype(jnp.bfloat16)
    ew = ew_ref[...].astype(jnp.bfloat16)
    y = jax.lax.dot_general(x, ew, (((1,), (1,)), ((), ())),
                            preferred_element_type=jnp.float32)
    out_ref[...] = ((y + eb_ref[...]) * s_col)[None]


def _make_sc_routing(T, experts, capacity):
    epw = experts // _NW
    n_vec = T // _LANES

    mesh = plsc.VectorSubcoreMesh(core_axis_name="c", subcore_axis_name="s",
                                  num_cores=_NUM_CORES)

    @functools.partial(
        pl.kernel,
        mesh=mesh,
        compiler_params=pltpu.CompilerParams(needs_layout_passes=False),
        out_type=jax.ShapeDtypeStruct((_NW, T), jnp.float32),
        scratch_types=[
            pltpu.VMEM((T,), jnp.int32),
            pltpu.VMEM((T,), jnp.float32),
            pltpu.VMEM((T,), jnp.float32),
        ],
    )
    def sc_routing(idx_hbm, gate_hbm, out_hbm, idx_v, gate_v, acc_v):
        c = lax.axis_index("c")
        s = lax.axis_index("s")
        wid = c * _NUM_SUBCORES + s
        e0v = jnp.full((_LANES,), wid * epw, jnp.int32)
        e1v = e0v + jnp.full((_LANES,), 1, jnp.int32)
        capv = jnp.full((_LANES,), capacity, jnp.int32)
        zerov = jnp.zeros((_LANES,), jnp.float32)
        onei = jnp.full((_LANES,), 1, jnp.int32)
        zeroi = jnp.zeros((_LANES,), jnp.int32)

        pltpu.sync_copy(idx_hbm, idx_v)
        pltpu.sync_copy(gate_hbm, gate_v)

        zero_base = jnp.zeros((_LANES,), jnp.int32)

        def body(i, carry):
            base0, base1 = carry
            off = pl.multiple_of(i * _LANES, _LANES)
            iv = idx_v[pl.ds(off, _LANES)]
            gv = gate_v[pl.ds(off, _LANES)]

            m0 = iv == e0v
            c0 = plsc.cumsum(jnp.where(m0, onei, zeroi))
            k0 = m0 & ((base0 + c0) <= capv)
            base0 = base0 + plsc.all_reduce_population_count(m0)

            m1 = iv == e1v
            c1 = plsc.cumsum(jnp.where(m1, onei, zeroi))
            k1 = m1 & ((base1 + c1) <= capv)
            base1 = base1 + plsc.all_reduce_population_count(m1)

            acc_v[pl.ds(off, _LANES)] = jnp.where(k0 | k1, gv, zerov)
            return base0, base1

        lax.fori_loop(0, n_vec, body, (zero_base, zero_base))
        pltpu.sync_copy(acc_v, out_hbm.at[wid])

    return sc_routing


def kernel(x, router_w, expert_w, expert_b):
    B, S, H = x.shape
    E = router_w.shape[1]
    T = B * S
    capacity = T // E * 2
    block = 256
    grid = T // block

    x3 = x.reshape(grid, block, H)
    eb = expert_b.reshape(1, H)

    idx3, gate3 = pl.pallas_call(
        functools.partial(_router_kernel, experts=E, block=block),
        grid=(grid,),
        in_specs=[
            pl.BlockSpec((1, block, H), lambda i: (i, 0, 0)),
            pl.BlockSpec((H, E), lambda i: (0, 0)),
        ],
        out_specs=[
            pl.BlockSpec((1, 1, block), lambda i: (i, 0, 0)),
            pl.BlockSpec((1, 1, block), lambda i: (i, 0, 0)),
        ],
        out_shape=[
            jax.ShapeDtypeStruct((grid, 1, block), jnp.int32),
            jax.ShapeDtypeStruct((grid, 1, block), jnp.float32),
        ],
    )(x3, router_w)

    sc_routing = _make_sc_routing(T, E, capacity)
    parts = sc_routing(idx3.reshape(T), gate3.reshape(T))

    out = pl.pallas_call(
        _expert_kernel,
        grid=(grid,),
        in_specs=[
            pl.BlockSpec((1, block, H), lambda i: (i, 0, 0)),
            pl.BlockSpec((H, H), lambda i: (0, 0)),
            pl.BlockSpec((1, H), lambda i: (0, 0)),
            pl.BlockSpec((_NW, block), lambda i: (0, i)),
        ],
        out_specs=pl.BlockSpec((1, block, H), lambda i: (i, 0, 0)),
        out_shape=jax.ShapeDtypeStruct((grid, block, H), jnp.float32),
    )(x3, expert_w, eb, parts)
    return out.reshape(B, S, H)


# speedup vs baseline: 1.0596x; 1.0030x over previous
import functools

import jax
import jax.numpy as jnp
from jax import lax
from jax.experimental import pallas as pl
from jax.experimental.pallas import tpu as pltpu
from jax.experimental.pallas import tpu_sc as plsc

_NUM_CORES = 2
_NUM_SUBCORES = 16
_LANES = 16
_NW = _NUM_CORES * _NUM_SUBCORES


def _router_kernel(x_ref, rw_ref, idx_ref, gate_ref, *, experts, block):
    x = x_ref[0]
    lt = jax.lax.dot_general(rw_ref[...], x, (((0,), (1,)), ((), ())),
                             preferred_element_type=jnp.float32)
    mx = jnp.max(lt, axis=0, keepdims=True)
    ex = jnp.exp(lt - mx)
    s = jnp.sum(ex, axis=0, keepdims=True)
    probs = ex / s
    pm = jnp.max(probs, axis=0, keepdims=True)
    row = jax.lax.broadcasted_iota(jnp.int32, probs.shape, 0)
    first = jnp.min(jnp.where(probs == pm, row, experts), axis=0,
                    keepdims=True)
    idx_ref[...] = first[None]
    gate_ref[...] = pm[None]


def _expert_kernel(x_ref, ew_ref, eb_ref, sp_ref, out_ref):
    sp = sp_ref[...]
    s_row = jnp.sum(sp, axis=0, keepdims=True)
    ones = jnp.ones((1, 1), dtype=jnp.float32)
    s_col = jax.lax.dot_general(s_row, ones, (((0,), (0,)), ((), ())),
                                preferred_element_type=jnp.float32)
    x = x_ref[0]
    y = jax.lax.dot_general(x, ew_ref[...], (((1,), (1,)), ((), ())),
                            preferred_element_type=jnp.float32)
    out_ref[...] = ((y + eb_ref[...]) * s_col)[None]


def _make_sc_routing(T, experts, capacity):
    epw = experts // _NW
    n_vec = T // _LANES

    mesh = plsc.VectorSubcoreMesh(core_axis_name="c", subcore_axis_name="s",
                                  num_cores=_NUM_CORES)

    @functools.partial(
        pl.kernel,
        mesh=mesh,
        compiler_params=pltpu.CompilerParams(needs_layout_passes=False),
        out_type=jax.ShapeDtypeStruct((_NW, T), jnp.float32),
        scratch_types=[
            pltpu.VMEM((T,), jnp.int32),
            pltpu.VMEM((T,), jnp.float32),
            pltpu.VMEM((T,), jnp.float32),
        ],
    )
    def sc_routing(idx_hbm, gate_hbm, out_hbm, idx_v, gate_v, acc_v):
        c = lax.axis_index("c")
        s = lax.axis_index("s")
        wid = c * _NUM_SUBCORES + s
        e0v = jnp.full((_LANES,), wid * epw, jnp.int32)
        e1v = e0v + jnp.full((_LANES,), 1, jnp.int32)
        capv = jnp.full((_LANES,), capacity, jnp.int32)
        zerov = jnp.zeros((_LANES,), jnp.float32)
        onei = jnp.full((_LANES,), 1, jnp.int32)
        zeroi = jnp.zeros((_LANES,), jnp.int32)

        pltpu.sync_copy(idx_hbm, idx_v)
        pltpu.sync_copy(gate_hbm, gate_v)

        zero_base = jnp.zeros((_LANES,), jnp.int32)

        def body(i, carry):
            base0, base1 = carry
            off = pl.multiple_of(i * _LANES, _LANES)
            iv = idx_v[pl.ds(off, _LANES)]
            gv = gate_v[pl.ds(off, _LANES)]

            m0 = iv == e0v
            c0 = plsc.cumsum(jnp.where(m0, onei, zeroi))
            k0 = m0 & ((base0 + c0) <= capv)
            base0 = base0 + plsc.all_reduce_population_count(m0)

            m1 = iv == e1v
            c1 = plsc.cumsum(jnp.where(m1, onei, zeroi))
            k1 = m1 & ((base1 + c1) <= capv)
            base1 = base1 + plsc.all_reduce_population_count(m1)

            acc_v[pl.ds(off, _LANES)] = jnp.where(k0 | k1, gv, zerov)
            return base0, base1

        lax.fori_loop(0, n_vec, body, (zero_base, zero_base))
        pltpu.sync_copy(acc_v, out_hbm.at[wid])

    return sc_routing


def kernel(x, router_w, expert_w, expert_b):
    B, S, H = x.shape
    E = router_w.shape[1]
    T = B * S
    capacity = T // E * 2
    block = 256
    grid = T // block

    eb = expert_b.reshape(1, H)
    x4 = x.reshape(1, T, H)

    idx3, gate3 = pl.pallas_call(
        functools.partial(_router_kernel, experts=E, block=block),
        grid=(grid,),
        in_specs=[
            pl.BlockSpec((1, block, H), lambda i: (0, i, 0)),
            pl.BlockSpec((H, E), lambda i: (0, 0)),
        ],
        out_specs=[
            pl.BlockSpec((1, 1, block), lambda i: (i, 0, 0)),
            pl.BlockSpec((1, 1, block), lambda i: (i, 0, 0)),
        ],
        out_shape=[
            jax.ShapeDtypeStruct((grid, 1, block), jnp.int32),
            jax.ShapeDtypeStruct((grid, 1, block), jnp.float32),
        ],
    )(x4, router_w)

    sc_routing = _make_sc_routing(T, E, capacity)
    parts = sc_routing(idx3.reshape(T), gate3.reshape(T))

    out = pl.pallas_call(
        _expert_kernel,
        grid=(grid,),
        in_specs=[
            pl.BlockSpec((1, block, H), lambda i: (0, i, 0)),
            pl.BlockSpec((H, H), lambda i: (0, 0)),
            pl.BlockSpec((1, H), lambda i: (0, 0)),
            pl.BlockSpec((_NW, block), lambda i: (0, i)),
        ],
        out_specs=pl.BlockSpec((1, block, H), lambda i: (0, i, 0)),
        out_shape=jax.ShapeDtypeStruct((1, T, H), jnp.float32),
    )(x4, expert_w, eb, parts)
    return out.reshape(B, S, H)


# speedup vs baseline: 1.1931x; 1.1260x over previous
import functools

import jax
import jax.numpy as jnp
from jax import lax
from jax.experimental import pallas as pl
from jax.experimental.pallas import tpu as pltpu
from jax.experimental.pallas import tpu_sc as plsc

_NUM_CORES = 2
_NUM_SUBCORES = 16
_LANES = 16
_NW = _NUM_CORES * _NUM_SUBCORES


def _router_kernel(x_ref, rw_ref, idx_ref, gate_ref, *, experts, block):
    x = x_ref[0]
    lt = jax.lax.dot_general(rw_ref[...], x, (((0,), (1,)), ((), ())),
                             preferred_element_type=jnp.float32)
    mx = jnp.max(lt, axis=0, keepdims=True)
    ex = jnp.exp(lt - mx)
    s = jnp.sum(ex, axis=0, keepdims=True)
    probs = ex / s
    pm = jnp.max(probs, axis=0, keepdims=True)
    row = jax.lax.broadcasted_iota(jnp.int32, probs.shape, 0)
    first = jnp.min(jnp.where(probs == pm, row, experts), axis=0,
                    keepdims=True)
    idx_ref[...] = first[None]
    gate_ref[...] = pm[None]


def _expert_kernel(x_ref, ew_ref, eb_ref, sp_ref, out_ref):
    sp = sp_ref[...]
    s_row = jnp.sum(sp, axis=0, keepdims=True)
    ones = jnp.ones((1, 1), dtype=jnp.float32)
    s_col = jax.lax.dot_general(s_row, ones, (((0,), (0,)), ((), ())),
                                preferred_element_type=jnp.float32)
    x = x_ref[0]
    y = jax.lax.dot_general(x, ew_ref[...], (((1,), (1,)), ((), ())),
                            preferred_element_type=jnp.float32)
    out_ref[...] = ((y + eb_ref[...]) * s_col)[None]


def _make_sc_routing(T, experts, capacity):
    epw = experts // _NW
    n_vec = T // _LANES

    mesh = plsc.VectorSubcoreMesh(core_axis_name="c", subcore_axis_name="s",
                                  num_cores=_NUM_CORES)

    @functools.partial(
        pl.kernel,
        mesh=mesh,
        compiler_params=pltpu.CompilerParams(needs_layout_passes=False),
        out_type=jax.ShapeDtypeStruct((_NW, T), jnp.float32),
        scratch_types=[
            pltpu.VMEM((T,), jnp.int32),
            pltpu.VMEM((T,), jnp.float32),
            pltpu.VMEM((T,), jnp.float32),
        ],
    )
    def sc_routing(idx_hbm, gate_hbm, out_hbm, idx_v, gate_v, acc_v):
        c = lax.axis_index("c")
        s = lax.axis_index("s")
        wid = c * _NUM_SUBCORES + s
        e0v = jnp.full((_LANES,), wid * epw, jnp.int32)
        e1v = e0v + jnp.full((_LANES,), 1, jnp.int32)
        capv = jnp.full((_LANES,), capacity, jnp.int32)
        zerov = jnp.zeros((_LANES,), jnp.float32)
        onei = jnp.full((_LANES,), 1, jnp.int32)
        zeroi = jnp.zeros((_LANES,), jnp.int32)

        pltpu.sync_copy(idx_hbm, idx_v)
        pltpu.sync_copy(gate_hbm, gate_v)

        zero_base = jnp.zeros((_LANES,), jnp.int32)

        def body(i, carry):
            base0, base1 = carry
            off = pl.multiple_of(i * _LANES, _LANES)
            iv = idx_v[pl.ds(off, _LANES)]
            gv = gate_v[pl.ds(off, _LANES)]

            m0 = iv == e0v
            c0 = plsc.cumsum(jnp.where(m0, onei, zeroi))
            k0 = m0 & ((base0 + c0) <= capv)
            base0 = base0 + plsc.all_reduce_population_count(m0)

            m1 = iv == e1v
            c1 = plsc.cumsum(jnp.where(m1, onei, zeroi))
            k1 = m1 & ((base1 + c1) <= capv)
            base1 = base1 + plsc.all_reduce_population_count(m1)

            acc_v[pl.ds(off, _LANES)] = jnp.where(k0 | k1, gv, zerov)
            return base0, base1

        lax.fori_loop(0, n_vec, body, (zero_base, zero_base))
        pltpu.sync_copy(acc_v, out_hbm.at[wid])

    return sc_routing


def kernel(x, router_w, expert_w, expert_b):
    B, S, H = x.shape
    E = router_w.shape[1]
    T = B * S
    capacity = T // E * 2
    block = 512
    grid = T // block

    eb = expert_b.reshape(1, H)
    x4 = x.reshape(1, T, H)

    idx3, gate3 = pl.pallas_call(
        functools.partial(_router_kernel, experts=E, block=block),
        grid=(grid,),
        in_specs=[
            pl.BlockSpec((1, block, H), lambda i: (0, i, 0)),
            pl.BlockSpec((H, E), lambda i: (0, 0)),
        ],
        out_specs=[
            pl.BlockSpec((1, 1, block), lambda i: (i, 0, 0)),
            pl.BlockSpec((1, 1, block), lambda i: (i, 0, 0)),
        ],
        out_shape=[
            jax.ShapeDtypeStruct((grid, 1, block), jnp.int32),
            jax.ShapeDtypeStruct((grid, 1, block), jnp.float32),
        ],
    )(x4, router_w)

    sc_routing = _make_sc_routing(T, E, capacity)
    parts = sc_routing(idx3.reshape(T), gate3.reshape(T))

    out = pl.pallas_call(
        _expert_kernel,
        grid=(grid,),
        in_specs=[
            pl.BlockSpec((1, block, H), lambda i: (0, i, 0)),
            pl.BlockSpec((H, H), lambda i: (0, 0)),
            pl.BlockSpec((1, H), lambda i: (0, 0)),
            pl.BlockSpec((_NW, block), lambda i: (0, i)),
        ],
        out_specs=pl.BlockSpec((1, block, H), lambda i: (0, i, 0)),
        out_shape=jax.ShapeDtypeStruct((1, T, H), jnp.float32),
    )(x4, expert_w, eb, parts)
    return out.reshape(B, S, H)


# speedup vs baseline: 1.2646x; 1.0599x over previous
import functools

import jax
import jax.numpy as jnp
from jax import lax
from jax.experimental import pallas as pl
from jax.experimental.pallas import tpu as pltpu
from jax.experimental.pallas import tpu_sc as plsc

_NUM_CORES = 2
_NUM_SUBCORES = 16
_LANES = 16
_NW = _NUM_CORES * _NUM_SUBCORES


def _router_kernel(x_ref, rw_ref, idx_ref, gate_ref, *, experts, block):
    x = x_ref[0]
    lt = jax.lax.dot_general(rw_ref[...], x, (((0,), (1,)), ((), ())),
                             preferred_element_type=jnp.float32)
    mx = jnp.max(lt, axis=0, keepdims=True)
    ex = jnp.exp(lt - mx)
    s = jnp.sum(ex, axis=0, keepdims=True)
    probs = ex / s
    pm = jnp.max(probs, axis=0, keepdims=True)
    row = jax.lax.broadcasted_iota(jnp.int32, probs.shape, 0)
    first = jnp.min(jnp.where(probs == pm, row, experts), axis=0,
                    keepdims=True)
    idx_ref[...] = first[None]
    gate_ref[...] = pm[None]


def _expert_kernel(x_ref, ew_ref, eb_ref, sp_ref, out_ref):
    sp = sp_ref[...]
    s_row = jnp.sum(sp, axis=0, keepdims=True)
    ones = jnp.ones((1, 1), dtype=jnp.float32)
    s_col = jax.lax.dot_general(s_row, ones, (((0,), (0,)), ((), ())),
                                preferred_element_type=jnp.float32)
    x = x_ref[0]
    y = jax.lax.dot_general(x, ew_ref[...], (((1,), (1,)), ((), ())),
                            preferred_element_type=jnp.float32)
    out_ref[...] = ((y + eb_ref[...]) * s_col)[None]


def _make_sc_routing(T, experts, capacity):
    epw = experts // _NW
    n_vec = T // _LANES

    mesh = plsc.VectorSubcoreMesh(core_axis_name="c", subcore_axis_name="s",
                                  num_cores=_NUM_CORES)

    @functools.partial(
        pl.kernel,
        mesh=mesh,
        compiler_params=pltpu.CompilerParams(needs_layout_passes=False),
        out_type=jax.ShapeDtypeStruct((_NW, T), jnp.float32),
        scratch_types=[
            pltpu.VMEM((T,), jnp.int32),
            pltpu.VMEM((T,), jnp.float32),
            pltpu.VMEM((T,), jnp.float32),
        ],
    )
    def sc_routing(idx_hbm, gate_hbm, out_hbm, idx_v, gate_v, acc_v):
        c = lax.axis_index("c")
        s = lax.axis_index("s")
        wid = c * _NUM_SUBCORES + s
        e0v = jnp.full((_LANES,), wid * epw, jnp.int32)
        e1v = e0v + jnp.full((_LANES,), 1, jnp.int32)
        capv = jnp.full((_LANES,), capacity, jnp.int32)
        zerov = jnp.zeros((_LANES,), jnp.float32)
        onei = jnp.full((_LANES,), 1, jnp.int32)
        zeroi = jnp.zeros((_LANES,), jnp.int32)

        pltpu.sync_copy(idx_hbm, idx_v)
        pltpu.sync_copy(gate_hbm, gate_v)

        zero_base = jnp.zeros((_LANES,), jnp.int32)

        def body(i, carry):
            base0, base1 = carry
            off = pl.multiple_of(i * _LANES, _LANES)
            iv = idx_v[pl.ds(off, _LANES)]
            gv = gate_v[pl.ds(off, _LANES)]

            m0 = iv == e0v
            c0 = plsc.cumsum(jnp.where(m0, onei, zeroi))
            k0 = m0 & ((base0 + c0) <= capv)
            base0 = base0 + plsc.all_reduce_population_count(m0)

            m1 = iv == e1v
            c1 = plsc.cumsum(jnp.where(m1, onei, zeroi))
            k1 = m1 & ((base1 + c1) <= capv)
            base1 = base1 + plsc.all_reduce_population_count(m1)

            acc_v[pl.ds(off, _LANES)] = jnp.where(k0 | k1, gv, zerov)
            return base0, base1

        lax.fori_loop(0, n_vec, body, (zero_base, zero_base))
        pltpu.sync_copy(acc_v, out_hbm.at[wid])

    return sc_routing


def kernel(x, router_w, expert_w, expert_b):
    B, S, H = x.shape
    E = router_w.shape[1]
    T = B * S
    capacity = T // E * 2
    block = 1024
    grid = T // block

    eb = expert_b.reshape(1, H)
    x4 = x.reshape(1, T, H)

    idx3, gate3 = pl.pallas_call(
        functools.partial(_router_kernel, experts=E, block=block),
        grid=(grid,),
        in_specs=[
            pl.BlockSpec((1, block, H), lambda i: (0, i, 0)),
            pl.BlockSpec((H, E), lambda i: (0, 0)),
        ],
        out_specs=[
            pl.BlockSpec((1, 1, block), lambda i: (i, 0, 0)),
            pl.BlockSpec((1, 1, block), lambda i: (i, 0, 0)),
        ],
        out_shape=[
            jax.ShapeDtypeStruct((grid, 1, block), jnp.int32),
            jax.ShapeDtypeStruct((grid, 1, block), jnp.float32),
        ],
    )(x4, router_w)

    sc_routing = _make_sc_routing(T, E, capacity)
    parts = sc_routing(idx3.reshape(T), gate3.reshape(T))

    out = pl.pallas_call(
        _expert_kernel,
        grid=(grid,),
        in_specs=[
            pl.BlockSpec((1, block, H), lambda i: (0, i, 0)),
            pl.BlockSpec((H, H), lambda i: (0, 0)),
            pl.BlockSpec((1, H), lambda i: (0, 0)),
            pl.BlockSpec((_NW, block), lambda i: (0, i)),
        ],
        out_specs=pl.BlockSpec((1, block, H), lambda i: (0, i, 0)),
        out_shape=jax.ShapeDtypeStruct((1, T, H), jnp.float32),
    )(x4, expert_w, eb, parts)
    return out.reshape(B, S, H)
